# Initial kernel scaffold; baseline (speedup 1.0000x reference)
import jax
import jax.numpy as jnp
from jax.experimental import pallas as pl


def kernel(x, W):
    raise NotImplementedError("write your pallas kernel here")


# speedup vs baseline: 1.7206x; 1.7206x over previous
import functools

import jax
import jax.numpy as jnp
from jax.experimental import pallas as pl
from jax.experimental.pallas import tpu as pltpu

N_TOKENS = 32768
D_MODEL = 768
NUM_EXPERTS = 64
TOP_K = 2

BLOCK_T = 1024


def _router_kernel(x_ref, w_ref, idx_ref, gate_ref, probs_ref):
    x = x_ref[...]
    w = w_ref[...]
    logits = jnp.dot(x, w, preferred_element_type=jnp.float32)
    m = jnp.max(logits, axis=1, keepdims=True)
    e = jnp.exp(logits - m)
    denom = jnp.sum(e, axis=1, keepdims=True)
    probs = e / denom
    probs_ref[...] = probs

    col = jax.lax.broadcasted_iota(jnp.int32, probs.shape, 1)
    p1 = jnp.max(probs, axis=1, keepdims=True)
    i1 = jnp.min(jnp.where(probs == p1, col, NUM_EXPERTS), axis=1, keepdims=True)
    masked = jnp.where(col == i1, -jnp.inf, probs)
    p2 = jnp.max(masked, axis=1, keepdims=True)
    i2 = jnp.min(jnp.where(masked == p2, col, NUM_EXPERTS), axis=1, keepdims=True)

    idx_ref[...] = jnp.concatenate([i1, i2], axis=1)
    gate_ref[...] = jnp.concatenate([p1, p2], axis=1)


@jax.jit
def kernel(x, W):
    n_tokens = x.shape[0]
    grid = (n_tokens // BLOCK_T,)
    idx, gates, probs = pl.pallas_call(
        _router_kernel,
        grid=grid,
        in_specs=[
            pl.BlockSpec((BLOCK_T, D_MODEL), lambda i: (i, 0)),
            pl.BlockSpec((D_MODEL, NUM_EXPERTS), lambda i: (0, 0)),
        ],
        out_specs=[
            pl.BlockSpec((BLOCK_T, TOP_K), lambda i: (i, 0)),
            pl.BlockSpec((BLOCK_T, TOP_K), lambda i: (i, 0)),
            pl.BlockSpec((BLOCK_T, NUM_EXPERTS), lambda i: (i, 0)),
        ],
        out_shape=[
            jax.ShapeDtypeStruct((n_tokens, TOP_K), jnp.int32),
            jax.ShapeDtypeStruct((n_tokens, TOP_K), jnp.float32),
            jax.ShapeDtypeStruct((n_tokens, NUM_EXPERTS), jnp.float32),
        ],
        compiler_params=pltpu.CompilerParams(
            dimension_semantics=("arbitrary",),
        ),
    )(x, W)
    return idx, gates, probs


# speedup vs baseline: 1.9483x; 1.1323x over previous
import functools

import jax
import jax.numpy as jnp
from jax.experimental import pallas as pl
from jax.experimental.pallas import tpu as pltpu

N_TOKENS = 32768
D_MODEL = 768
NUM_EXPERTS = 64
TOP_K = 2

BLOCK_T = 2048


def _router_kernel(x_ref, w_ref, idx_ref, gate_ref, probs_ref):
    x = x_ref[...]
    w = w_ref[...]
    logits = jnp.dot(x, w, preferred_element_type=jnp.float32)
    m = jnp.max(logits, axis=1, keepdims=True)
    e = jnp.exp(logits - m)
    denom = jnp.sum(e, axis=1, keepdims=True)
    probs = e / denom
    probs_ref[...] = probs

    col = jax.lax.broadcasted_iota(jnp.int32, probs.shape, 1)
    p1 = jnp.max(probs, axis=1, keepdims=True)
    i1 = jnp.min(jnp.where(probs == p1, col, NUM_EXPERTS), axis=1, keepdims=True)
    masked = jnp.where(col == i1, -jnp.inf, probs)
    p2 = jnp.max(masked, axis=1, keepdims=True)
    i2 = jnp.min(jnp.where(masked == p2, col, NUM_EXPERTS), axis=1, keepdims=True)

    idx_ref[...] = jnp.concatenate([i1, i2], axis=1)
    gate_ref[...] = jnp.concatenate([p1, p2], axis=1)


@jax.jit
def kernel(x, W):
    n_tokens = x.shape[0]
    grid = (n_tokens // BLOCK_T,)
    idx, gates, probs = pl.pallas_call(
        _router_kernel,
        grid=grid,
        in_specs=[
            pl.BlockSpec((BLOCK_T, D_MODEL), lambda i: (i, 0)),
            pl.BlockSpec((D_MODEL, NUM_EXPERTS), lambda i: (0, 0)),
        ],
        out_specs=[
            pl.BlockSpec((BLOCK_T, TOP_K), lambda i: (i, 0)),
            pl.BlockSpec((BLOCK_T, TOP_K), lambda i: (i, 0)),
            pl.BlockSpec((BLOCK_T, NUM_EXPERTS), lambda i: (i, 0)),
        ],
        out_shape=[
            jax.ShapeDtypeStruct((n_tokens, TOP_K), jnp.int32),
            jax.ShapeDtypeStruct((n_tokens, TOP_K), jnp.float32),
            jax.ShapeDtypeStruct((n_tokens, NUM_EXPERTS), jnp.float32),
        ],
        compiler_params=pltpu.CompilerParams(
            dimension_semantics=("arbitrary",),
        ),
    )(x, W)
    return idx, gates, probs


# speedup vs baseline: 2.0756x; 1.0653x over previous
import functools

import jax
import jax.numpy as jnp
from jax.experimental import pallas as pl
from jax.experimental.pallas import tpu as pltpu

N_TOKENS = 32768
D_MODEL = 768
NUM_EXPERTS = 64
TOP_K = 2

BLOCK_T = 4096


def _router_kernel(x_ref, w_ref, idx_ref, gate_ref, probs_ref):
    x = x_ref[...]
    w = w_ref[...]
    logits = jnp.dot(x, w, preferred_element_type=jnp.float32)
    m = jnp.max(logits, axis=1, keepdims=True)
    e = jnp.exp(logits - m)
    denom = jnp.sum(e, axis=1, keepdims=True)
    probs = e / denom
    probs_ref[...] = probs

    col = jax.lax.broadcasted_iota(jnp.int32, probs.shape, 1)
    p1 = jnp.max(probs, axis=1, keepdims=True)
    i1 = jnp.min(jnp.where(probs == p1, col, NUM_EXPERTS), axis=1, keepdims=True)
    masked = jnp.where(col == i1, -jnp.inf, probs)
    p2 = jnp.max(masked, axis=1, keepdims=True)
    i2 = jnp.min(jnp.where(masked == p2, col, NUM_EXPERTS), axis=1, keepdims=True)

    idx_ref[...] = jnp.concatenate([i1, i2], axis=1)
    gate_ref[...] = jnp.concatenate([p1, p2], axis=1)


@jax.jit
def kernel(x, W):
    n_tokens = x.shape[0]
    grid = (n_tokens // BLOCK_T,)
    idx, gates, probs = pl.pallas_call(
        _router_kernel,
        grid=grid,
        in_specs=[
            pl.BlockSpec((BLOCK_T, D_MODEL), lambda i: (i, 0)),
            pl.BlockSpec((D_MODEL, NUM_EXPERTS), lambda i: (0, 0)),
        ],
        out_specs=[
            pl.BlockSpec((BLOCK_T, TOP_K), lambda i: (i, 0)),
            pl.BlockSpec((BLOCK_T, TOP_K), lambda i: (i, 0)),
            pl.BlockSpec((BLOCK_T, NUM_EXPERTS), lambda i: (i, 0)),
        ],
        out_shape=[
            jax.ShapeDtypeStruct((n_tokens, TOP_K), jnp.int32),
            jax.ShapeDtypeStruct((n_tokens, TOP_K), jnp.float32),
            jax.ShapeDtypeStruct((n_tokens, NUM_EXPERTS), jnp.float32),
        ],
        compiler_params=pltpu.CompilerParams(
            dimension_semantics=("arbitrary",),
        ),
    )(x, W)
    return idx, gates, probs


# speedup vs baseline: 2.0796x; 1.0019x over previous
import jax
import jax.numpy as jnp
from jax import lax
from jax.experimental import pallas as pl
from jax.experimental.pallas import tpu as pltpu

N_TOKENS = 32768
D_MODEL = 768
NUM_EXPERTS = 64
TOP_K = 2

BLOCK_T = 2048
NBUF = 4


def _router_kernel(x_hbm, w_ref, idx_ref, gate_ref, probs_ref, xbuf, sems):
    i = pl.program_id(0)
    nb = pl.num_programs(0)

    def copy_in(chunk, slot):
        return pltpu.make_async_copy(
            x_hbm.at[pl.ds(chunk * BLOCK_T, BLOCK_T), :],
            xbuf.at[slot],
            sems.at[slot],
        )

    @pl.when(i == 0)
    def _prime():
        for b in range(NBUF):
            copy_in(b, b).start()

    slot = lax.rem(i, NBUF)
    copy_in(i, slot).wait()

    x = xbuf[slot]
    w = w_ref[...]
    logits = jnp.dot(x, w, preferred_element_type=jnp.float32)
    m = jnp.max(logits, axis=1, keepdims=True)
    e = jnp.exp(logits - m)
    denom = jnp.sum(e, axis=1, keepdims=True)
    probs = e / denom
    probs_ref[...] = probs

    col = jax.lax.broadcasted_iota(jnp.int32, probs.shape, 1)
    p1 = jnp.max(probs, axis=1, keepdims=True)
    i1 = jnp.min(jnp.where(probs == p1, col, NUM_EXPERTS), axis=1, keepdims=True)
    masked = jnp.where(col == i1, -jnp.inf, probs)
    p2 = jnp.max(masked, axis=1, keepdims=True)
    i2 = jnp.min(jnp.where(masked == p2, col, NUM_EXPERTS), axis=1, keepdims=True)

    idx_ref[...] = jnp.concatenate([i1, i2], axis=1)
    gate_ref[...] = jnp.concatenate([p1, p2], axis=1)

    nxt = i + NBUF

    @pl.when(nxt < nb)
    def _refill():
        copy_in(nxt, slot).start()


@jax.jit
def kernel(x, W):
    n_tokens = x.shape[0]
    grid = (n_tokens // BLOCK_T,)
    idx, gates, probs = pl.pallas_call(
        _router_kernel,
        grid=grid,
        in_specs=[
            pl.BlockSpec(memory_space=pl.ANY),
            pl.BlockSpec((D_MODEL, NUM_EXPERTS), lambda i: (0, 0)),
        ],
        out_specs=[
            pl.BlockSpec((BLOCK_T, TOP_K), lambda i: (i, 0)),
            pl.BlockSpec((BLOCK_T, TOP_K), lambda i: (i, 0)),
            pl.BlockSpec((BLOCK_T, NUM_EXPERTS), lambda i: (i, 0)),
        ],
        out_shape=[
            jax.ShapeDtypeStruct((n_tokens, TOP_K), jnp.int32),
            jax.ShapeDtypeStruct((n_tokens, TOP_K), jnp.float32),
            jax.ShapeDtypeStruct((n_tokens, NUM_EXPERTS), jnp.float32),
        ],
        scratch_shapes=[
            pltpu.VMEM((NBUF, BLOCK_T, D_MODEL), jnp.float32),
            pltpu.SemaphoreType.DMA((NBUF,)),
        ],
        compiler_params=pltpu.CompilerParams(
            dimension_semantics=("arbitrary",),
        ),
    )(x, W)
    return idx, gates, probs
